# baseline (speedup 1.0000x reference)
import numpy as np

import jax
import jax.numpy as jnp
from jax.experimental import pallas as pl
from jax.experimental.pallas import tpu as pltpu

_TB = 512


def _conv1_tables():
    k = np.arange(256)
    di = k // 32
    jj = k % 32
    n = np.arange(672)
    rp = n // 336
    r = (n % 336) // 168
    jpar = (n % 168) // 84
    jh = (n % 84) // 6
    co = n % 6
    cr = 2 * rp + r
    j = 2 * jh + jpar
    a = di[:, None] - cr[None, :]
    b = jj[:, None] - j[None, :]
    mask = (a >= 0) & (a < 5) & (b >= 0) & (b < 5)
    return (mask, np.broadcast_to(co[None, :], (256, 672)),
            np.clip(a, 0, 4), np.clip(b, 0, 4))


def _conv2_tables():
    k = np.arange(504)
    di = k // 84
    jj = (k % 84) // 6
    ci = k % 6
    n = np.arange(320)
    r = n // 160
    jpar = (n % 160) // 80
    jh = (n % 80) // 16
    co = n % 16
    j = 2 * jh + jpar
    a = di[:, None] - r[None, :]
    b = jj[:, None] - j[None, :]
    mask = (a >= 0) & (a < 5) & (b >= 0) & (b < 5)
    return (mask, np.broadcast_to(co[None, :], (504, 320)),
            np.broadcast_to(ci[:, None], (504, 320)),
            np.clip(a, 0, 4), np.clip(b, 0, 4))


_C1_MASK, _C1_CO, _C1_A, _C1_B = _conv1_tables()
_C2_MASK, _C2_CO, _C2_CI, _C2_A, _C2_B = _conv2_tables()

_L = np.arange(400)
_FC1_PERM = (_L % 16) * 25 + (_L // 80) * 5 + ((_L % 80) // 16)
_B1_IDX = np.arange(84) % 6
_B2_IDX = np.arange(80) % 16


def _fused_kernel(x_ref, w1_ref, b1_ref, w2_ref, b2_ref,
                  fw1_ref, fb1_ref, fw2_ref, fb2_ref, fw3_ref, fb3_ref,
                  o_ref, xb_ref, a1_ref, a2_ref):
    f32 = jnp.float32
    bf = jnp.bfloat16
    xb_ref[...] = x_ref[...].astype(bf)

    for q in range(7):
        t = jnp.dot(xb_ref[:, 128 * q:128 * q + 256], w1_ref[...],
                    preferred_element_type=f32)
        r0 = jnp.maximum(t[:, 0:168], t[:, 168:336])
        r1 = jnp.maximum(t[:, 336:504], t[:, 504:672])
        c0 = jnp.maximum(r0[:, 0:84], r0[:, 84:168]) + b1_ref[...]
        c1 = jnp.maximum(r1[:, 0:84], r1[:, 84:168]) + b1_ref[...]
        a1_ref[:, 168 * q:168 * q + 84] = jnp.maximum(c0, 0.0).astype(bf)
        a1_ref[:, 168 * q + 84:168 * q + 168] = jnp.maximum(c1, 0.0).astype(bf)

    for p in range(5):
        u = jnp.dot(a1_ref[:, 168 * p:168 * p + 504], w2_ref[...],
                    preferred_element_type=f32)
        m = jnp.maximum(u[:, 0:160], u[:, 160:320])
        m = jnp.maximum(m[:, 0:80], m[:, 80:160]) + b2_ref[...]
        a2_ref[:, 80 * p:80 * p + 80] = jnp.maximum(m, 0.0).astype(bf)

    h = jnp.dot(a2_ref[...], fw1_ref[...], preferred_element_type=f32)
    h = jnp.maximum(h + fb1_ref[...], 0.0).astype(bf)
    h = jnp.dot(h, fw2_ref[...], preferred_element_type=f32)
    h = jnp.maximum(h + fb2_ref[...], 0.0).astype(bf)
    o = jnp.dot(h, fw3_ref[...], preferred_element_type=f32)
    o_ref[...] = o + fb3_ref[...]


def kernel(x, conv1_w, conv1_b, conv2_w, conv2_b,
           fc1_w, fc1_b, fc2_w, fc2_b, fc3_w, fc3_b):
    B = x.shape[0]
    b_pad = -(-B // _TB) * _TB
    x2d = x.reshape(B, 1024)
    if b_pad != B:
        x2d = jnp.pad(x2d, ((0, b_pad - B), (0, 0)))
    bf = jnp.bfloat16

    w1r = conv1_w.reshape(6, 5, 5)
    w1t = jnp.where(_C1_MASK, w1r[_C1_CO, _C1_A, _C1_B], 0.0).astype(bf)
    w2r = conv2_w.reshape(16, 6, 5, 5)
    w2t = jnp.where(_C2_MASK, w2r[_C2_CO, _C2_CI, _C2_A, _C2_B],
                    0.0).astype(bf)
    b1v = conv1_b[_B1_IDX].reshape(1, 84)
    b2v = conv2_b[_B2_IDX].reshape(1, 80)
    fw1 = fc1_w[:, _FC1_PERM].T.astype(bf)
    fb1 = fc1_b.reshape(1, 128)
    fw2 = fc2_w.T.astype(bf)
    fb2 = fc2_b.reshape(1, 128)
    fw3 = fc3_w.T.astype(bf)
    fb3 = fc3_b.reshape(1, 10)

    def whole(a):
        zeros = (0,) * a.ndim
        return pl.BlockSpec(a.shape, lambda i, z=zeros: z)

    flops = 2 * B * (6 * 25 * 28 * 28 + 16 * 150 * 100
                     + 400 * 120 + 120 * 84 + 84 * 10)
    bytes_accessed = 4 * int(x.size) + B * 10 * 4

    out = pl.pallas_call(
        _fused_kernel,
        out_shape=jax.ShapeDtypeStruct((b_pad, 10), jnp.float32),
        grid=(b_pad // _TB,),
        in_specs=[pl.BlockSpec((_TB, 1024), lambda i: (i, 0)),
                  whole(w1t), whole(b1v), whole(w2t), whole(b2v),
                  whole(fw1), whole(fb1), whole(fw2), whole(fb2),
                  whole(fw3), whole(fb3)],
        out_specs=pl.BlockSpec((_TB, 10), lambda i: (i, 0)),
        scratch_shapes=[pltpu.VMEM((_TB, 1024), jnp.bfloat16),
                        pltpu.VMEM((_TB, 1176), jnp.bfloat16),
                        pltpu.VMEM((_TB, 400), jnp.bfloat16)],
        compiler_params=pltpu.CompilerParams(
            dimension_semantics=("parallel",)),
        cost_estimate=pl.CostEstimate(flops=flops, transcendentals=0,
                                      bytes_accessed=bytes_accessed),
    )(x2d, w1t, b1v, w2t, b2v, fw1, fb1, fw2, fb2, fw3, fb3)
    return out[:B]


# speedup vs baseline: 17.5925x; 17.5925x over previous
import numpy as np

import jax
import jax.numpy as jnp
from jax.experimental import pallas as pl
from jax.experimental.pallas import tpu as pltpu

_TB = 512


def _conv1_masks():
    n = np.arange(672)
    rp, r = n // 336, (n % 336) // 168
    jpar, jh = (n % 168) // 84, (n % 84) // 6
    cr = 2 * rp + r
    j = 2 * jh + jpar
    u = np.zeros((5, 8, 672), np.float32)
    v = np.zeros((5, 32, 672), np.float32)
    for a in range(5):
        u[a] = np.arange(8)[:, None] == (cr + a)[None, :]
        v[a] = np.arange(32)[:, None] == (j + a)[None, :]
    return u, v


def _conv2_masks():
    n = np.arange(320)
    r = n // 160
    jpar, jh = (n % 160) // 80, (n % 80) // 16
    j = 2 * jh + jpar
    u = np.zeros((5, 6, 320), np.float32)
    v = np.zeros((5, 14, 320), np.float32)
    for a in range(5):
        u[a] = np.arange(6)[:, None] == (r + a)[None, :]
        v[a] = np.arange(14)[:, None] == (j + a)[None, :]
    return u, v


_U1, _V1 = _conv1_masks()
_U2, _V2 = _conv2_masks()


def _build_toeplitz(conv1_w, conv2_w):
    w1r = conv1_w.reshape(6, 5, 5)
    acc1 = jnp.zeros((8, 32, 672), jnp.float32)
    for a in range(5):
        for b in range(5):
            vab = jnp.broadcast_to(w1r[:, a, b], (112, 6)).reshape(672)
            acc1 += _U1[a][:, None, :] * (_V1[b][None, :, :] * vab)
    w1t = acc1.reshape(256, 672)

    w2r = conv2_w.reshape(16, 6, 5, 5)
    acc2 = jnp.zeros((6, 14, 6, 320), jnp.float32)
    for a in range(5):
        for b in range(5):
            t = jnp.broadcast_to(w2r[:, :, a, b].T.reshape(6, 1, 16),
                                 (6, 20, 16)).reshape(6, 320)
            acc2 += (_U2[a][:, None, None, :] * _V2[b][None, :, None, :]
                     * t[None, None, :, :])
    w2t = acc2.reshape(504, 320)
    return w1t, w2t


def _fused_kernel(x_ref, w1_ref, b1_ref, w2_ref, b2_ref,
                  fw1_ref, fb1_ref, fw2_ref, fb2_ref, fw3_ref, fb3_ref,
                  o_ref, xb_ref, a1_ref, a2_ref):
    f32 = jnp.float32
    bf = jnp.bfloat16
    xb_ref[...] = x_ref[...].astype(bf)

    for q in range(7):
        t = jnp.dot(xb_ref[:, 128 * q:128 * q + 256], w1_ref[...],
                    preferred_element_type=f32)
        r0 = jnp.maximum(t[:, 0:168], t[:, 168:336])
        r1 = jnp.maximum(t[:, 336:504], t[:, 504:672])
        c0 = jnp.maximum(r0[:, 0:84], r0[:, 84:168]) + b1_ref[...]
        c1 = jnp.maximum(r1[:, 0:84], r1[:, 84:168]) + b1_ref[...]
        a1_ref[:, 168 * q:168 * q + 84] = jnp.maximum(c0, 0.0).astype(bf)
        a1_ref[:, 168 * q + 84:168 * q + 168] = jnp.maximum(c1, 0.0).astype(bf)

    for p in range(5):
        u = jnp.dot(a1_ref[:, 168 * p:168 * p + 504], w2_ref[...],
                    preferred_element_type=f32)
        m = jnp.maximum(u[:, 0:160], u[:, 160:320])
        m = jnp.maximum(m[:, 0:80], m[:, 80:160]) + b2_ref[...]
        a2_ref[:, 80 * p:80 * p + 80] = jnp.maximum(m, 0.0).astype(bf)

    h = jnp.dot(a2_ref[...], fw1_ref[...], preferred_element_type=f32)
    h = jnp.maximum(h + fb1_ref[...], 0.0).astype(bf)
    h = jnp.dot(h, fw2_ref[...], preferred_element_type=f32)
    h = jnp.maximum(h + fb2_ref[...], 0.0).astype(bf)
    o = jnp.dot(h, fw3_ref[...], preferred_element_type=f32)
    o_ref[...] = o + fb3_ref[...]


def kernel(x, conv1_w, conv1_b, conv2_w, conv2_b,
           fc1_w, fc1_b, fc2_w, fc2_b, fc3_w, fc3_b):
    B = x.shape[0]
    b_pad = -(-B // _TB) * _TB
    x2d = x.reshape(B, 1024)
    if b_pad != B:
        x2d = jnp.pad(x2d, ((0, b_pad - B), (0, 0)))
    bf = jnp.bfloat16

    w1t, w2t = _build_toeplitz(conv1_w, conv2_w)
    w1t = w1t.astype(bf)
    w2t = w2t.astype(bf)
    b1v = jnp.broadcast_to(conv1_b, (14, 6)).reshape(1, 84)
    b2v = jnp.broadcast_to(conv2_b, (5, 16)).reshape(1, 80)
    fw1 = fc1_w.reshape(128, 16, 5, 5).transpose(2, 3, 1, 0) \
               .reshape(400, 128).astype(bf)
    fb1 = fc1_b.reshape(1, 128)
    fw2 = fc2_w.T.astype(bf)
    fb2 = fc2_b.reshape(1, 128)
    fw3 = fc3_w.T.astype(bf)
    fb3 = fc3_b.reshape(1, 10)

    def whole(a):
        zeros = (0,) * a.ndim
        return pl.BlockSpec(a.shape, lambda i, z=zeros: z)

    flops = 2 * B * (6 * 25 * 28 * 28 + 16 * 150 * 100
                     + 400 * 120 + 120 * 84 + 84 * 10)
    bytes_accessed = 4 * int(x.size) + B * 10 * 4

    out = pl.pallas_call(
        _fused_kernel,
        out_shape=jax.ShapeDtypeStruct((b_pad, 10), jnp.float32),
        grid=(b_pad // _TB,),
        in_specs=[pl.BlockSpec((_TB, 1024), lambda i: (i, 0)),
                  whole(w1t), whole(b1v), whole(w2t), whole(b2v),
                  whole(fw1), whole(fb1), whole(fw2), whole(fb2),
                  whole(fw3), whole(fb3)],
        out_specs=pl.BlockSpec((_TB, 10), lambda i: (i, 0)),
        scratch_shapes=[pltpu.VMEM((_TB, 1024), jnp.bfloat16),
                        pltpu.VMEM((_TB, 1176), jnp.bfloat16),
                        pltpu.VMEM((_TB, 400), jnp.bfloat16)],
        compiler_params=pltpu.CompilerParams(
            dimension_semantics=("parallel",)),
        cost_estimate=pl.CostEstimate(flops=flops, transcendentals=0,
                                      bytes_accessed=bytes_accessed),
    )(x2d, w1t, b1v, w2t, b2v, fw1, fb1, fw2, fb2, fw3, fb3)
    return out[:B]


# speedup vs baseline: 26.7183x; 1.5187x over previous
import numpy as np

import jax
import jax.numpy as jnp
from jax.experimental import pallas as pl
from jax.experimental.pallas import tpu as pltpu

_TB = 512


def _conv1_masks():
    n = np.arange(672)
    rp, r = n // 336, (n % 336) // 168
    jpar, jh = (n % 168) // 84, (n % 84) // 6
    cr = 2 * rp + r
    j = 2 * jh + jpar
    u = np.zeros((5, 8, 672), np.float32)
    v = np.zeros((5, 32, 672), np.float32)
    for a in range(5):
        u[a] = np.arange(8)[:, None] == (cr + a)[None, :]
        v[a] = np.arange(32)[:, None] == (j + a)[None, :]
    return u, v


def _conv2_masks():
    n = np.arange(320)
    r = n // 160
    jpar, jh = (n % 160) // 80, (n % 80) // 16
    j = 2 * jh + jpar
    u = np.zeros((5, 6, 320), np.float32)
    v = np.zeros((5, 14, 320), np.float32)
    for a in range(5):
        u[a] = np.arange(6)[:, None] == (r + a)[None, :]
        v[a] = np.arange(14)[:, None] == (j + a)[None, :]
    return u, v


_U1, _V1 = _conv1_masks()
_U2, _V2 = _conv2_masks()


def _build_toeplitz(conv1_w, conv2_w):
    w1r = conv1_w.reshape(6, 5, 5)
    acc1 = jnp.zeros((8, 32, 672), jnp.float32)
    for a in range(5):
        pa = jnp.zeros((32, 672), jnp.float32)
        for b in range(5):
            vab = jnp.broadcast_to(w1r[:, a, b], (112, 6)).reshape(672)
            pa += _V1[b] * vab
        acc1 += _U1[a][:, None, :] * pa[None, :, :]
    w1t = acc1.reshape(256, 672)

    w2r = conv2_w.reshape(16, 6, 5, 5)
    acc2 = jnp.zeros((6, 14, 6, 320), jnp.float32)
    for a in range(5):
        qa = jnp.zeros((14, 6, 320), jnp.float32)
        for b in range(5):
            t = jnp.broadcast_to(w2r[:, :, a, b].T.reshape(6, 1, 16),
                                 (6, 20, 16)).reshape(6, 320)
            qa += _V2[b][:, None, :] * t[None, :, :]
        acc2 += _U2[a][:, None, None, :] * qa[None, :, :, :]
    w2t = acc2.reshape(504, 320)
    return w1t, w2t


def _fused_kernel(x_ref, w1_ref, b1_ref, w2_ref, b2_ref,
                  fw1_ref, fb1_ref, fw2_ref, fb2_ref, fw3_ref, fb3_ref,
                  o_ref, xb_ref, a1_ref, a2_ref):
    f32 = jnp.float32
    bf = jnp.bfloat16
    xb_ref[...] = x_ref[...].astype(bf)

    for q in range(7):
        t = jnp.dot(xb_ref[:, 128 * q:128 * q + 256], w1_ref[...],
                    preferred_element_type=f32).astype(bf)
        r0 = jnp.maximum(t[:, 0:168], t[:, 168:336])
        r1 = jnp.maximum(t[:, 336:504], t[:, 504:672])
        c0 = jnp.maximum(r0[:, 0:84], r0[:, 84:168]) + b1_ref[...]
        c1 = jnp.maximum(r1[:, 0:84], r1[:, 84:168]) + b1_ref[...]
        a1_ref[:, 168 * q:168 * q + 84] = jnp.maximum(c0, 0)
        a1_ref[:, 168 * q + 84:168 * q + 168] = jnp.maximum(c1, 0)

    for p in range(5):
        u = jnp.dot(a1_ref[:, 168 * p:168 * p + 504], w2_ref[...],
                    preferred_element_type=f32).astype(bf)
        m = jnp.maximum(u[:, 0:160], u[:, 160:320])
        m = jnp.maximum(m[:, 0:80], m[:, 80:160]) + b2_ref[...]
        a2_ref[:, 80 * p:80 * p + 80] = jnp.maximum(m, 0)

    h = jnp.dot(a2_ref[...], fw1_ref[...], preferred_element_type=f32)
    h = jnp.maximum(h + fb1_ref[...], 0.0).astype(bf)
    h = jnp.dot(h, fw2_ref[...], preferred_element_type=f32)
    h = jnp.maximum(h + fb2_ref[...], 0.0).astype(bf)
    o = jnp.dot(h, fw3_ref[...], preferred_element_type=f32)
    o_ref[...] = o + fb3_ref[...]


def kernel(x, conv1_w, conv1_b, conv2_w, conv2_b,
           fc1_w, fc1_b, fc2_w, fc2_b, fc3_w, fc3_b):
    B = x.shape[0]
    b_pad = -(-B // _TB) * _TB
    x2d = x.reshape(B, 1024)
    if b_pad != B:
        x2d = jnp.pad(x2d, ((0, b_pad - B), (0, 0)))
    bf = jnp.bfloat16

    w1t, w2t = _build_toeplitz(conv1_w, conv2_w)
    w1t = w1t.astype(bf)
    w2t = w2t.astype(bf)
    b1v = jnp.broadcast_to(conv1_b, (14, 6)).reshape(1, 84).astype(bf)
    b2v = jnp.broadcast_to(conv2_b, (5, 16)).reshape(1, 80).astype(bf)
    fw1 = fc1_w.reshape(128, 16, 5, 5).transpose(2, 3, 1, 0) \
               .reshape(400, 128).astype(bf)
    fb1 = fc1_b.reshape(1, 128)
    fw2 = fc2_w.T.astype(bf)
    fb2 = fc2_b.reshape(1, 128)
    fw3 = fc3_w.T.astype(bf)
    fb3 = fc3_b.reshape(1, 10)

    def whole(a):
        zeros = (0,) * a.ndim
        return pl.BlockSpec(a.shape, lambda *_, z=zeros: z)

    flops = 2 * B * (6 * 25 * 28 * 28 + 16 * 150 * 100
                     + 400 * 120 + 120 * 84 + 84 * 10)
    bytes_accessed = 4 * int(x.size) + B * 10 * 4

    out = pl.pallas_call(
        _fused_kernel,
        out_shape=jax.ShapeDtypeStruct((b_pad, 10), jnp.float32),
        grid=(b_pad // _TB,),
        in_specs=[pl.BlockSpec((_TB, 1024), lambda i: (i, 0)),
                  whole(w1t), whole(b1v), whole(w2t), whole(b2v),
                  whole(fw1), whole(fb1), whole(fw2), whole(fb2),
                  whole(fw3), whole(fb3)],
        out_specs=pl.BlockSpec((_TB, 10), lambda i: (i, 0)),
        scratch_shapes=[pltpu.VMEM((_TB, 1024), jnp.bfloat16),
                        pltpu.VMEM((_TB, 1176), jnp.bfloat16),
                        pltpu.VMEM((_TB, 400), jnp.bfloat16)],
        compiler_params=pltpu.CompilerParams(
            dimension_semantics=("arbitrary",)),
        cost_estimate=pl.CostEstimate(flops=flops, transcendentals=0,
                                      bytes_accessed=bytes_accessed),
    )(x2d, w1t, b1v, w2t, b2v, fw1, fb1, fw2, fb2, fw3, fb3)
    return out[:B]


# speedup vs baseline: 27.3661x; 1.0242x over previous
import numpy as np

import jax
import jax.numpy as jnp
from jax.experimental import pallas as pl
from jax.experimental.pallas import tpu as pltpu

_TB = 1024


def _conv1_masks():
    n = np.arange(672)
    rp, r = n // 336, (n % 336) // 168
    jpar, jh = (n % 168) // 84, (n % 84) // 6
    cr = 2 * rp + r
    j = 2 * jh + jpar
    u = np.zeros((5, 8, 672), np.float32)
    v = np.zeros((5, 32, 672), np.float32)
    for a in range(5):
        u[a] = np.arange(8)[:, None] == (cr + a)[None, :]
        v[a] = np.arange(32)[:, None] == (j + a)[None, :]
    return u, v


def _conv2_masks():
    n = np.arange(320)
    r = n // 160
    jpar, jh = (n % 160) // 80, (n % 80) // 16
    j = 2 * jh + jpar
    u = np.zeros((5, 6, 320), np.float32)
    v = np.zeros((5, 14, 320), np.float32)
    for a in range(5):
        u[a] = np.arange(6)[:, None] == (r + a)[None, :]
        v[a] = np.arange(14)[:, None] == (j + a)[None, :]
    return u, v


_U1, _V1 = _conv1_masks()
_U2, _V2 = _conv2_masks()


def _build_toeplitz(conv1_w, conv2_w):
    w1r = conv1_w.reshape(6, 5, 5)
    acc1 = jnp.zeros((8, 32, 672), jnp.float32)
    for a in range(5):
        pa = jnp.zeros((32, 672), jnp.float32)
        for b in range(5):
            vab = jnp.broadcast_to(w1r[:, a, b], (112, 6)).reshape(672)
            pa += _V1[b] * vab
        acc1 += _U1[a][:, None, :] * pa[None, :, :]
    w1t = acc1.reshape(256, 672)

    w2r = conv2_w.reshape(16, 6, 5, 5)
    acc2 = jnp.zeros((6, 14, 6, 320), jnp.float32)
    for a in range(5):
        qa = jnp.zeros((14, 6, 320), jnp.float32)
        for b in range(5):
            t = jnp.broadcast_to(w2r[:, :, a, b].T.reshape(6, 1, 16),
                                 (6, 20, 16)).reshape(6, 320)
            qa += _V2[b][:, None, :] * t[None, :, :]
        acc2 += _U2[a][:, None, None, :] * qa[None, :, :, :]
    w2t = acc2.reshape(504, 320)
    return w1t, w2t


def _fused_kernel(x_ref, w1_ref, b1_ref, w2_ref, b2_ref,
                  fw1_ref, fb1_ref, fw2_ref, fb2_ref, fw3_ref, fb3_ref,
                  o_ref, xb_ref, a1_ref, a2_ref):
    f32 = jnp.float32
    bf = jnp.bfloat16
    xb_ref[...] = x_ref[...].astype(bf)

    for q in range(7):
        t = jnp.dot(xb_ref[:, 128 * q:128 * q + 256], w1_ref[...],
                    preferred_element_type=f32).astype(bf)
        r0 = jnp.maximum(t[:, 0:168], t[:, 168:336])
        r1 = jnp.maximum(t[:, 336:504], t[:, 504:672])
        c0 = jnp.maximum(r0[:, 0:84], r0[:, 84:168]) + b1_ref[...]
        c1 = jnp.maximum(r1[:, 0:84], r1[:, 84:168]) + b1_ref[...]
        a1_ref[:, 168 * q:168 * q + 84] = jnp.maximum(c0, 0)
        a1_ref[:, 168 * q + 84:168 * q + 168] = jnp.maximum(c1, 0)

    for p in range(5):
        u = jnp.dot(a1_ref[:, 168 * p:168 * p + 504], w2_ref[...],
                    preferred_element_type=f32).astype(bf)
        m = jnp.maximum(u[:, 0:160], u[:, 160:320])
        m = jnp.maximum(m[:, 0:80], m[:, 80:160]) + b2_ref[...]
        a2_ref[:, 80 * p:80 * p + 80] = jnp.maximum(m, 0)

    h = jnp.dot(a2_ref[...], fw1_ref[...], preferred_element_type=f32)
    h = jnp.maximum(h + fb1_ref[...], 0.0).astype(bf)
    h = jnp.dot(h, fw2_ref[...], preferred_element_type=f32)
    h = jnp.maximum(h + fb2_ref[...], 0.0).astype(bf)
    o = jnp.dot(h, fw3_ref[...], preferred_element_type=f32)
    o_ref[...] = o + fb3_ref[...]


def kernel(x, conv1_w, conv1_b, conv2_w, conv2_b,
           fc1_w, fc1_b, fc2_w, fc2_b, fc3_w, fc3_b):
    B = x.shape[0]
    b_pad = -(-B // _TB) * _TB
    x2d = x.reshape(B, 1024)
    if b_pad != B:
        x2d = jnp.pad(x2d, ((0, b_pad - B), (0, 0)))
    bf = jnp.bfloat16

    w1t, w2t = _build_toeplitz(conv1_w, conv2_w)
    w1t = w1t.astype(bf)
    w2t = w2t.astype(bf)
    b1v = jnp.broadcast_to(conv1_b, (14, 6)).reshape(1, 84).astype(bf)
    b2v = jnp.broadcast_to(conv2_b, (5, 16)).reshape(1, 80).astype(bf)
    fw1 = fc1_w.reshape(128, 16, 5, 5).transpose(2, 3, 1, 0) \
               .reshape(400, 128).astype(bf)
    fb1 = fc1_b.reshape(1, 128)
    fw2 = fc2_w.T.astype(bf)
    fb2 = fc2_b.reshape(1, 128)
    fw3 = fc3_w.T.astype(bf)
    fb3 = fc3_b.reshape(1, 10)

    def whole(a):
        zeros = (0,) * a.ndim
        return pl.BlockSpec(a.shape, lambda *_, z=zeros: z)

    flops = 2 * B * (6 * 25 * 28 * 28 + 16 * 150 * 100
                     + 400 * 120 + 120 * 84 + 84 * 10)
    bytes_accessed = 4 * int(x.size) + B * 10 * 4

    out = pl.pallas_call(
        _fused_kernel,
        out_shape=jax.ShapeDtypeStruct((b_pad, 10), jnp.float32),
        grid=(b_pad // _TB,),
        in_specs=[pl.BlockSpec((_TB, 1024), lambda i: (i, 0)),
                  whole(w1t), whole(b1v), whole(w2t), whole(b2v),
                  whole(fw1), whole(fb1), whole(fw2), whole(fb2),
                  whole(fw3), whole(fb3)],
        out_specs=pl.BlockSpec((_TB, 10), lambda i: (i, 0)),
        scratch_shapes=[pltpu.VMEM((_TB, 1024), jnp.bfloat16),
                        pltpu.VMEM((_TB, 1176), jnp.bfloat16),
                        pltpu.VMEM((_TB, 400), jnp.bfloat16)],
        compiler_params=pltpu.CompilerParams(
            dimension_semantics=("arbitrary",)),
        cost_estimate=pl.CostEstimate(flops=flops, transcendentals=0,
                                      bytes_accessed=bytes_accessed),
    )(x2d, w1t, b1v, w2t, b2v, fw1, fb1, fw2, fb2, fw3, fb3)
    return out[:B]


# speedup vs baseline: 33.0648x; 1.2082x over previous
import numpy as np

import jax
import jax.numpy as jnp
from jax.experimental import pallas as pl
from jax.experimental.pallas import tpu as pltpu

_TB = 1024


def _conv1_masks():
    n = np.arange(1024)
    cr = n // 256
    jpar = (n % 256) // 128
    l = n % 128
    jh, co = l // 6, l % 6
    valid = l < 84
    j = 2 * jh + jpar
    u = np.zeros((5, 8, 1024), np.float32)
    v = np.zeros((5, 32, 1024), np.float32)
    for a in range(5):
        u[a] = np.arange(8)[:, None] == (cr + a)[None, :]
        v[a] = (np.arange(32)[:, None] == (j + a)[None, :]) & valid[None, :]
    com = (np.arange(6)[:, None] == co[None, :]) & valid[None, :]
    return u, v, com.astype(np.float32)


def _conv2_masks():
    n = np.arange(512)
    r = n // 256
    jpar = (n % 256) // 128
    l = n % 128
    jh, co = l // 16, l % 16
    valid = l < 80
    j = 2 * jh + jpar
    u = np.zeros((5, 6, 512), np.float32)
    v = np.zeros((5, 14, 512), np.float32)
    for a in range(5):
        u[a] = np.arange(6)[:, None] == (r + a)[None, :]
        v[a] = (np.arange(14)[:, None] == (j + a)[None, :]) & valid[None, :]
    com = (np.arange(16)[:, None] == co[None, :]) & valid[None, :]
    return u, v, com.astype(np.float32)


_U1, _V1, _CO1 = _conv1_masks()
_U2, _V2, _CO2 = _conv2_masks()


def _build_toeplitz(conv1_w, conv2_w):
    vw1 = jnp.dot(conv1_w.reshape(6, 25).T, _CO1)
    acc1 = jnp.zeros((8, 32, 1024), jnp.float32)
    for a in range(5):
        pa = jnp.zeros((32, 1024), jnp.float32)
        for b in range(5):
            pa += _V1[b] * vw1[a * 5 + b]
        acc1 += _U1[a][:, None, :] * pa[None]
    w1t = acc1.reshape(256, 1024)

    vw2 = jnp.dot(conv2_w.reshape(16, 150).T, _CO2)
    acc2 = jnp.zeros((6, 14, 6, 512), jnp.float32)
    for a in range(5):
        qa = jnp.zeros((14, 6, 512), jnp.float32)
        for b in range(5):
            qa += _V2[b][:, None, :] * vw2[a * 5 + b::25][None, :, :]
        acc2 += _U2[a][:, None, None, :] * qa[None]
    w2t = jnp.pad(acc2.reshape(6, 84, 512),
                  ((0, 0), (0, 44), (0, 0))).reshape(768, 512)
    return w1t, w2t


def _fused_kernel(x_ref, w1_ref, b1_ref, w2_ref, b2_ref,
                  fw1_ref, fb1_ref, fw2_ref, fb2_ref, fw3_ref, fb3_ref,
                  o_ref, xb_ref, a1_ref, a2_ref):
    f32 = jnp.float32
    bf = jnp.bfloat16
    xb_ref[...] = x_ref[...].astype(bf)

    for q in range(7):
        t = jnp.dot(xb_ref[:, 128 * q:128 * q + 256], w1_ref[...],
                    preferred_element_type=f32).astype(bf)
        m0 = jnp.maximum(t[:, 0:256], t[:, 256:512])
        m1 = jnp.maximum(t[:, 512:768], t[:, 768:1024])
        c0 = jnp.maximum(m0[:, 0:128], m0[:, 128:256]) + b1_ref[...]
        c1 = jnp.maximum(m1[:, 0:128], m1[:, 128:256]) + b1_ref[...]
        a1_ref[:, 256 * q:256 * q + 128] = jnp.maximum(c0, 0)
        a1_ref[:, 256 * q + 128:256 * q + 256] = jnp.maximum(c1, 0)

    for p in range(5):
        u = jnp.dot(a1_ref[:, 256 * p:256 * p + 768], w2_ref[...],
                    preferred_element_type=f32).astype(bf)
        m = jnp.maximum(u[:, 0:256], u[:, 256:512])
        c = jnp.maximum(m[:, 0:128], m[:, 128:256]) + b2_ref[...]
        a2_ref[:, 128 * p:128 * p + 128] = jnp.maximum(c, 0)

    h = jnp.dot(a2_ref[...], fw1_ref[...], preferred_element_type=f32)
    h = jnp.maximum(h + fb1_ref[...], 0.0).astype(bf)
    h = jnp.dot(h, fw2_ref[...], preferred_element_type=f32)
    h = jnp.maximum(h + fb2_ref[...], 0.0).astype(bf)
    o = jnp.dot(h, fw3_ref[...], preferred_element_type=f32)
    o_ref[...] = o + fb3_ref[...]


def kernel(x, conv1_w, conv1_b, conv2_w, conv2_b,
           fc1_w, fc1_b, fc2_w, fc2_b, fc3_w, fc3_b):
    B = x.shape[0]
    b_pad = -(-B // _TB) * _TB
    x2d = x.reshape(B, 1024)
    if b_pad != B:
        x2d = jnp.pad(x2d, ((0, b_pad - B), (0, 0)))
    bf = jnp.bfloat16

    w1t, w2t = _build_toeplitz(conv1_w, conv2_w)
    w1t = w1t.astype(bf)
    w2t = w2t.astype(bf)
    b1v = jnp.pad(jnp.broadcast_to(conv1_b, (14, 6)).reshape(1, 84),
                  ((0, 0), (0, 44))).astype(bf)
    b2v = jnp.pad(jnp.broadcast_to(conv2_b, (5, 16)).reshape(1, 80),
                  ((0, 0), (0, 48))).astype(bf)
    fw1 = jnp.pad(fc1_w.reshape(128, 16, 5, 5).transpose(2, 3, 1, 0)
                  .reshape(5, 80, 128), ((0, 0), (0, 48), (0, 0))) \
             .reshape(640, 128).astype(bf)
    fb1 = fc1_b.reshape(1, 128)
    fw2 = fc2_w.T.astype(bf)
    fb2 = fc2_b.reshape(1, 128)
    fw3 = fc3_w.T.astype(bf)
    fb3 = fc3_b.reshape(1, 10)

    def whole(a):
        zeros = (0,) * a.ndim
        return pl.BlockSpec(a.shape, lambda *_, z=zeros: z)

    flops = 2 * B * (6 * 25 * 28 * 28 + 16 * 150 * 100
                     + 400 * 120 + 120 * 84 + 84 * 10)
    bytes_accessed = 4 * int(x.size) + B * 10 * 4

    out = pl.pallas_call(
        _fused_kernel,
        out_shape=jax.ShapeDtypeStruct((b_pad, 10), jnp.float32),
        grid=(b_pad // _TB,),
        in_specs=[pl.BlockSpec((_TB, 1024), lambda i: (i, 0)),
                  whole(w1t), whole(b1v), whole(w2t), whole(b2v),
                  whole(fw1), whole(fb1), whole(fw2), whole(fb2),
                  whole(fw3), whole(fb3)],
        out_specs=pl.BlockSpec((_TB, 10), lambda i: (i, 0)),
        scratch_shapes=[pltpu.VMEM((_TB, 1024), jnp.bfloat16),
                        pltpu.VMEM((_TB, 1792), jnp.bfloat16),
                        pltpu.VMEM((_TB, 640), jnp.bfloat16)],
        compiler_params=pltpu.CompilerParams(
            dimension_semantics=("arbitrary",)),
        cost_estimate=pl.CostEstimate(flops=flops, transcendentals=0,
                                      bytes_accessed=bytes_accessed),
    )(x2d, w1t, b1v, w2t, b2v, fw1, fb1, fw2, fb2, fw3, fb3)
    return out[:B]
